# baseline (speedup 1.0000x reference)
import functools

import jax
import jax.numpy as jnp
from jax.experimental import pallas as pl
from jax.experimental.pallas import tpu as pltpu
from jax.experimental.pallas import tpu_sc as plsc

_N = 10000
_E = 320000
_D = 128
_R = 200
_EPS = 1e-5

_NC = 2
_NS = 16
_NW = _NC * _NS
_EPT = _E // _NW
_CH = 50
_NCHUNK = _EPT // _CH
_NPAD = 10112
_RPW = _NPAD // _NS
_ZR = 8


def _tables_body(x_ref, w_ref, out_ref):
    out_ref[...] = jnp.dot(x_ref[...], w_ref[0],
                           preferred_element_type=jnp.float32,
                           precision=jax.lax.Precision.HIGHEST)


def _node_tables(x, in_w, out_w):
    w_stack = jnp.stack([in_w, out_w])
    nb = 10
    blk = _N // nb
    return pl.pallas_call(
        _tables_body,
        grid=(2, nb),
        in_specs=[
            pl.BlockSpec((blk, _D), lambda w, i: (i, 0)),
            pl.BlockSpec((1, _D, _D), lambda w, i: (w, 0, 0)),
        ],
        out_specs=pl.BlockSpec((blk, _D), lambda w, i: (w * nb + i, 0)),
        out_shape=jax.ShapeDtypeStruct((2 * _N, _D), jnp.float32),
    )(x, w_stack)


def _rel_body(rel_ref, inw_ref, outw_ref, wrel_ref, rcomb_ref, relout_ref):
    r = rel_ref[...]
    hi = jax.lax.Precision.HIGHEST
    rcomb_ref[pl.ds(0, _R), :] = jnp.dot(r, inw_ref[...],
                                         preferred_element_type=jnp.float32,
                                         precision=hi)
    rcomb_ref[pl.ds(_R, _R), :] = jnp.dot(r, outw_ref[...],
                                          preferred_element_type=jnp.float32,
                                          precision=hi)
    relout_ref[...] = jnp.dot(r, wrel_ref[...],
                              preferred_element_type=jnp.float32,
                              precision=hi)


def _rel_tables(rel, in_w, out_w, w_rel):
    return pl.pallas_call(
        _rel_body,
        out_shape=(
            jax.ShapeDtypeStruct((2 * _R, _D), jnp.float32),
            jax.ShapeDtypeStruct((_R, _D), jnp.float32),
        ),
    )(rel, in_w, out_w, w_rel)


def _epilogue_body(p_ref, x_ref, lw_ref, lrel_ref, bias_ref, g_ref, b_ref,
                   out_ref):
    agg = p_ref[0, pl.ds(0, _N)] + p_ref[1, pl.ds(0, _N)]
    loop_term = jnp.dot(x_ref[...] - lrel_ref[...], lw_ref[...],
                        preferred_element_type=jnp.float32,
                        precision=jax.lax.Precision.HIGHEST)
    h = (agg + loop_term) * (1.0 / 3.0) + bias_ref[...]
    mean = jnp.mean(h, axis=0, keepdims=True)
    var = jnp.mean((h - mean) ** 2, axis=0, keepdims=True)
    h = (h - mean) / jnp.sqrt(var + _EPS) * g_ref[...] + b_ref[...]
    out_ref[...] = jnp.maximum(h, 0.0)


def _epilogue(partials, x, loop_w, loop_rel, bias, bn_gamma, bn_beta):
    return pl.pallas_call(
        _epilogue_body,
        out_shape=jax.ShapeDtypeStruct((_N, _D), jnp.float32),
    )(partials, x, loop_w, loop_rel.reshape(1, _D), bias.reshape(1, _D),
      bn_gamma.reshape(1, _D), bn_beta.reshape(1, _D))


def _sc_edge_scatter(tcomb, rcomb, srcp, typep, dst, norm):
    mesh = plsc.VectorSubcoreMesh(core_axis_name="c", subcore_axis_name="s")

    @functools.partial(
        pl.kernel,
        out_type=jax.ShapeDtypeStruct((_NC, _NPAD, _D), jnp.float32),
        mesh=mesh,
        scratch_types=[
            pltpu.VMEM_SHARED((_NPAD, _D), jnp.float32),
            pltpu.VMEM_SHARED((_R, _D), jnp.float32),
            pltpu.VMEM((_CH,), jnp.int32),
            pltpu.VMEM((_CH,), jnp.int32),
            pltpu.VMEM((_CH,), jnp.int32),
            pltpu.VMEM((_CH,), jnp.int32),
            pltpu.VMEM((_CH,), jnp.int32),
            pltpu.VMEM((_CH,), jnp.int32),
            pltpu.VMEM((_CH,), jnp.int32),
            pltpu.VMEM((_CH,), jnp.int32),
            pltpu.VMEM((_CH,), jnp.int32),
            pltpu.VMEM((_CH,), jnp.int32),
            pltpu.VMEM((_CH,), jnp.int32),
            pltpu.VMEM((_CH,), jnp.int32),
            pltpu.VMEM((_CH * 16,), jnp.float32),
            pltpu.VMEM((_CH * 16,), jnp.float32),
            pltpu.VMEM((_CH * 16,), jnp.float32),
            pltpu.VMEM((_CH * 16,), jnp.float32),
            pltpu.VMEM((_CH, _D), jnp.float32),
            pltpu.VMEM((_CH, _D), jnp.float32),
            pltpu.VMEM((_CH, _D), jnp.float32),
            pltpu.VMEM((_CH, _D), jnp.float32),
            pltpu.VMEM((_CH, _D), jnp.float32),
            pltpu.VMEM((_CH, _D), jnp.float32),
            pltpu.SemaphoreType.DMA,
            pltpu.SemaphoreType.DMA,
            pltpu.SemaphoreType.DMA,
            pltpu.SemaphoreType.DMA,
            pltpu.SemaphoreType.DMA,
            pltpu.SemaphoreType.DMA,
            pltpu.SemaphoreType.DMA,
            pltpu.SemaphoreType.DMA,
            pltpu.SemaphoreType.DMA,
            pltpu.SemaphoreType.DMA,
            pltpu.SemaphoreType.DMA,
            pltpu.SemaphoreType.DMA,
            pltpu.SemaphoreType.DMA,
            pltpu.SemaphoreType.DMA,
        ],
    )
    def k(t_hbm, r_hbm, src_hbm, typ_hbm, dst_hbm, nrm_hbm, out_hbm,
          acc, rsp, sv0, sv1, sv2, sv3, tv0, tv1, tv2, tv3, dv0, dv1, dv2, dv3,
          nv0, nv1, nv2, nv3, t0, t1, t2, t3, r0, r1,
          si0, si1, si2, si3, st0, st1, st2, st3, sr0, sr1,
          ss0, ss1, ss2, ss3):
        c = jax.lax.axis_index("c")
        s = jax.lax.axis_index("s")
        wid = c * _NS + s
        srcv = (sv0, sv1, sv2, sv3)
        typv = (tv0, tv1, tv2, tv3)
        dstv = (dv0, dv1, dv2, dv3)
        nrmv = (nv0, nv1, nv2, nv3)
        trow = (t0, t1, t2, t3)
        rrow = (r0, r1)
        semi = (si0, si1, si2, si3)
        semt = (st0, st1, st2, st3)
        semr = (sr0, sr1)
        sems = (ss0, ss1, ss2, ss3)

        zero16 = jnp.zeros((16,), jnp.float32)
        for i in range(_CH):
            for j in range(_D // 16):
                t0[i, pl.ds(j * 16, 16)] = zero16
        for i in range(13):
            pltpu.make_async_copy(
                t0.at[pl.ds(0, 48)],
                acc.at[pl.ds(s * _RPW + i * 48, 48)], si0).start()
        pltpu.make_async_copy(
            t0.at[pl.ds(0, 8)],
            acc.at[pl.ds(s * _RPW + 624, 8)], si0).start()
        for i in range(13):
            pltpu.make_async_copy(
                t0.at[pl.ds(0, 48)],
                acc.at[pl.ds(s * _RPW + i * 48, 48)], si0).wait()
        pltpu.make_async_copy(
            t0.at[pl.ds(0, 8)],
            acc.at[pl.ds(s * _RPW + 624, 8)], si0).wait()
        @pl.when(s == 0)
        def _():
            for i in range(_R // 40):
                pltpu.sync_copy(r_hbm.at[pl.ds(c * _R + i * 40, 40)],
                                r0.at[pl.ds(0, 40)])
                pltpu.sync_copy(r0.at[pl.ds(0, 40)],
                                rsp.at[pl.ds(i * 40, 40)])
        plsc.subcore_barrier()

        def idx_descs(j, b):
            return (
                pltpu.make_async_copy(src_hbm.at[wid, j], srcv[b], semi[b]),
                pltpu.make_async_copy(typ_hbm.at[wid, j], typv[b], semi[b]),
                pltpu.make_async_copy(dst_hbm.at[wid, j], dstv[b], semi[b]),
                pltpu.make_async_copy(nrm_hbm.at[wid, j], nrmv[b], semi[b]),
            )

        def gather_descs(b, b2):
            return (
                pltpu.make_async_copy(t_hbm.at[srcv[b]], trow[b], semt[b]),
                pltpu.make_async_copy(rsp.at[typv[b]], rrow[b2], semr[b2]),
            )

        def scat_desc(b):
            return pltpu.make_async_copy(trow[b], acc.at[dstv[b]], sems[b])

        def compute(b, b2):
            tb, rb, nb_ref = trow[b], rrow[b2], nrmv[b]

            def edge5(e5, _):
                for u in range(10):
                    e = e5 * 10 + u
                    nb = nb_ref[pl.ds(e * 16, 16)]
                    for jj in range(_D // 16):
                        t = tb[e, pl.ds(jj * 16, 16)]
                        r = rb[e, pl.ds(jj * 16, 16)]
                        tb[e, pl.ds(jj * 16, 16)] = (t - r) * nb
                return 0
            jax.lax.fori_loop(0, _CH // 10, edge5, 0)

        for d in idx_descs(0, 0):
            d.start()
        for d in idx_descs(1, 1):
            d.start()
        for d in idx_descs(0, 0):
            d.wait()
        for d in gather_descs(0, 0):
            d.start()

        def outer(kb, _):
            for u in range(4):
                kk = kb * 4 + u
                b = u
                b1 = (u + 1) % 4
                b2s = (u + 2) % 4
                @pl.when(kk >= 2)
                def _():
                    scat_desc(b2s).wait()
                @pl.when(kk + 2 < _NCHUNK)
                def _():
                    for d in idx_descs(kk + 2, b2s):
                        d.start()
                for d in gather_descs(b, u % 2):
                    d.wait()
                @pl.when(kk + 1 < _NCHUNK)
                def _():
                    for d in idx_descs(kk + 1, b1):
                        d.wait()
                    for d in gather_descs(b1, (u + 1) % 2):
                        d.start()
                compute(b, u % 2)
                scat_desc(b).start(add=True)
            return 0
        jax.lax.fori_loop(0, _NCHUNK // 4, outer, 0)

        scat_desc((_NCHUNK - 2) % 4).wait()
        scat_desc((_NCHUNK - 1) % 4).wait()

        plsc.subcore_barrier()
        pltpu.make_async_copy(acc.at[pl.ds(s * _RPW, _RPW)],
                              out_hbm.at[c, pl.ds(s * _RPW, _RPW)],
                              ss0).start()
        pltpu.make_async_copy(acc.at[pl.ds(s * _RPW, _RPW)],
                              out_hbm.at[c, pl.ds(s * _RPW, _RPW)],
                              ss0).wait()

    return k(tcomb, rcomb, srcp, typep, dst, norm)


def kernel(x, rel_repr, edge_index, edge_type, edge_norm,
           in_w, out_w, loop_w, w_rel, loop_rel, bias, bn_gamma, bn_beta):
    half = _E // 2
    src = edge_index[0].astype(jnp.int32)
    dst = edge_index[1].astype(jnp.int32)
    shift = (jnp.arange(_E, dtype=jnp.int32) >= half).astype(jnp.int32)
    srcp = (src + shift * _N).reshape(_NW, _NCHUNK, _CH)
    typep = edge_type.astype(jnp.int32).reshape(_NW, _NCHUNK, _CH)
    dst3 = dst.reshape(_NW, _NCHUNK, _CH)
    norm16 = jnp.reshape(
        jnp.broadcast_to(edge_norm[:, None], (_E, 16)),
        (_NW, _NCHUNK, _CH * 16))

    tcomb = _node_tables(x, in_w, out_w)
    rcomb, rel_out = _rel_tables(rel_repr, in_w, out_w, w_rel)
    partials = _sc_edge_scatter(tcomb, rcomb, srcp, typep, dst3, norm16)
    out = _epilogue(partials, x, loop_w, loop_rel, bias, bn_gamma, bn_beta)
    return out, rel_out


# speedup vs baseline: 1.0002x; 1.0002x over previous
import functools

import jax
import jax.numpy as jnp
from jax.experimental import pallas as pl
from jax.experimental.pallas import tpu as pltpu
from jax.experimental.pallas import tpu_sc as plsc

_N = 10000
_E = 320000
_D = 128
_R = 200
_EPS = 1e-5

_NC = 2
_NS = 16
_NW = _NC * _NS
_EPT = _E // _NW
_CH = 50
_NCHUNK = _EPT // _CH
_NPAD = 10112
_RPW = _NPAD // _NS
_ZR = 8


def _tables_body(x_ref, w_ref, out_ref):
    out_ref[...] = jnp.dot(x_ref[...], w_ref[0],
                           preferred_element_type=jnp.float32,
                           precision=jax.lax.Precision.HIGHEST)


def _node_tables(x, in_w, out_w):
    w_stack = jnp.stack([in_w, out_w])
    nb = 10
    blk = _N // nb
    return pl.pallas_call(
        _tables_body,
        grid=(2, nb),
        in_specs=[
            pl.BlockSpec((blk, _D), lambda w, i: (i, 0)),
            pl.BlockSpec((1, _D, _D), lambda w, i: (w, 0, 0)),
        ],
        out_specs=pl.BlockSpec((blk, _D), lambda w, i: (w * nb + i, 0)),
        out_shape=jax.ShapeDtypeStruct((2 * _N, _D), jnp.float32),
    )(x, w_stack)


def _rel_body(rel_ref, inw_ref, outw_ref, wrel_ref, rcomb_ref, relout_ref):
    r = rel_ref[...]
    hi = jax.lax.Precision.HIGHEST
    rcomb_ref[pl.ds(0, _R), :] = -jnp.dot(r, inw_ref[...],
                                          preferred_element_type=jnp.float32,
                                          precision=hi)
    rcomb_ref[pl.ds(_R, _R), :] = -jnp.dot(r, outw_ref[...],
                                           preferred_element_type=jnp.float32,
                                           precision=hi)
    relout_ref[...] = jnp.dot(r, wrel_ref[...],
                              preferred_element_type=jnp.float32,
                              precision=hi)


def _rel_tables(rel, in_w, out_w, w_rel):
    return pl.pallas_call(
        _rel_body,
        out_shape=(
            jax.ShapeDtypeStruct((2 * _R, _D), jnp.float32),
            jax.ShapeDtypeStruct((_R, _D), jnp.float32),
        ),
    )(rel, in_w, out_w, w_rel)


def _epilogue_body(p_ref, x_ref, lw_ref, lrel_ref, bias_ref, g_ref, b_ref,
                   out_ref):
    agg = p_ref[0, pl.ds(0, _N)] + p_ref[1, pl.ds(0, _N)]
    loop_term = jnp.dot(x_ref[...] - lrel_ref[...], lw_ref[...],
                        preferred_element_type=jnp.float32,
                        precision=jax.lax.Precision.HIGHEST)
    h = (agg + loop_term) * (1.0 / 3.0) + bias_ref[...]
    mean = jnp.mean(h, axis=0, keepdims=True)
    var = jnp.mean((h - mean) ** 2, axis=0, keepdims=True)
    h = (h - mean) / jnp.sqrt(var + _EPS) * g_ref[...] + b_ref[...]
    out_ref[...] = jnp.maximum(h, 0.0)


def _epilogue(partials, x, loop_w, loop_rel, bias, bn_gamma, bn_beta):
    return pl.pallas_call(
        _epilogue_body,
        out_shape=jax.ShapeDtypeStruct((_N, _D), jnp.float32),
    )(partials, x, loop_w, loop_rel.reshape(1, _D), bias.reshape(1, _D),
      bn_gamma.reshape(1, _D), bn_beta.reshape(1, _D))


def _sc_edge_scatter(tcomb, rcomb, srcp, typep, dst, norm):
    mesh = plsc.VectorSubcoreMesh(core_axis_name="c", subcore_axis_name="s")

    @functools.partial(
        pl.kernel,
        out_type=jax.ShapeDtypeStruct((_NC, _NPAD, _D), jnp.float32),
        mesh=mesh,
        scratch_types=[
            pltpu.VMEM_SHARED((_NPAD, _D), jnp.float32),
            pltpu.VMEM_SHARED((_R, _D), jnp.float32),
            pltpu.VMEM((_CH,), jnp.int32),
            pltpu.VMEM((_CH,), jnp.int32),
            pltpu.VMEM((_CH,), jnp.int32),
            pltpu.VMEM((_CH,), jnp.int32),
            pltpu.VMEM((_CH,), jnp.int32),
            pltpu.VMEM((_CH,), jnp.int32),
            pltpu.VMEM((_CH,), jnp.int32),
            pltpu.VMEM((_CH,), jnp.int32),
            pltpu.VMEM((_CH,), jnp.int32),
            pltpu.VMEM((_CH,), jnp.int32),
            pltpu.VMEM((_CH,), jnp.int32),
            pltpu.VMEM((_CH,), jnp.int32),
            pltpu.VMEM((_CH * 16,), jnp.float32),
            pltpu.VMEM((_CH * 16,), jnp.float32),
            pltpu.VMEM((_CH * 16,), jnp.float32),
            pltpu.VMEM((_CH * 16,), jnp.float32),
            pltpu.VMEM((_CH, _D), jnp.float32),
            pltpu.VMEM((_CH, _D), jnp.float32),
            pltpu.VMEM((_CH, _D), jnp.float32),
            pltpu.VMEM((_CH, _D), jnp.float32),
            pltpu.SemaphoreType.DMA,
            pltpu.SemaphoreType.DMA,
            pltpu.SemaphoreType.DMA,
            pltpu.SemaphoreType.DMA,
            pltpu.SemaphoreType.DMA,
            pltpu.SemaphoreType.DMA,
            pltpu.SemaphoreType.DMA,
            pltpu.SemaphoreType.DMA,
            pltpu.SemaphoreType.DMA,
            pltpu.SemaphoreType.DMA,
            pltpu.SemaphoreType.DMA,
            pltpu.SemaphoreType.DMA,
            pltpu.SemaphoreType.DMA,
            pltpu.SemaphoreType.DMA,
            pltpu.SemaphoreType.DMA,
            pltpu.SemaphoreType.DMA,
        ],
    )
    def k(t_hbm, r_hbm, src_hbm, typ_hbm, dst_hbm, nrm_hbm, out_hbm,
          acc, rsp, sv0, sv1, sv2, sv3, tv0, tv1, tv2, tv3, dv0, dv1, dv2, dv3,
          nv0, nv1, nv2, nv3, t0, t1, t2, t3,
          si0, si1, si2, si3, st0, st1, st2, st3, sd0, sd1, sd2, sd3,
          ss0, ss1, ss2, ss3):
        c = jax.lax.axis_index("c")
        s = jax.lax.axis_index("s")
        wid = c * _NS + s
        srcv = (sv0, sv1, sv2, sv3)
        typv = (tv0, tv1, tv2, tv3)
        dstv = (dv0, dv1, dv2, dv3)
        nrmv = (nv0, nv1, nv2, nv3)
        trow = (t0, t1, t2, t3)
        semi = (si0, si1, si2, si3)
        semt = (st0, st1, st2, st3)
        semd = (sd0, sd1, sd2, sd3)
        sems = (ss0, ss1, ss2, ss3)

        zero16 = jnp.zeros((16,), jnp.float32)
        for i in range(8):
            for j in range(_D // 16):
                t0[i, pl.ds(j * 16, 16)] = zero16
        nz = _RPW // 8
        for i in range(nz):
            pltpu.make_async_copy(
                t0.at[pl.ds(0, 8)],
                acc.at[pl.ds(s * _RPW + i * 8, 8)], si0).start()
        for i in range(nz):
            pltpu.make_async_copy(
                t0.at[pl.ds(0, 8)],
                acc.at[pl.ds(s * _RPW + i * 8, 8)], si0).wait()
        @pl.when(s == 0)
        def _():
            for i in range(_R // 40):
                pltpu.sync_copy(r_hbm.at[pl.ds(c * _R + i * 40, 40)],
                                t0.at[pl.ds(0, 40)])
                pltpu.sync_copy(t0.at[pl.ds(0, 40)],
                                rsp.at[pl.ds(i * 40, 40)])
        plsc.subcore_barrier()

        def idx3_descs(j, b):
            return (
                pltpu.make_async_copy(src_hbm.at[wid, j], srcv[b], semi[b]),
                pltpu.make_async_copy(typ_hbm.at[wid, j], typv[b], semi[b]),
                pltpu.make_async_copy(nrm_hbm.at[wid, j], nrmv[b], semi[b]),
            )

        def dst_desc(j, b):
            return pltpu.make_async_copy(dst_hbm.at[wid, j], dstv[b], semd[b])

        def t_desc(b):
            return pltpu.make_async_copy(t_hbm.at[srcv[b]], trow[b], semt[b])

        def radd_desc(b):
            return pltpu.make_async_copy(rsp.at[typv[b]], trow[b], semt[b])

        def scat_desc(b):
            return pltpu.make_async_copy(trow[b], acc.at[dstv[b]], sems[b])

        def compute(b):
            tb, nb_ref = trow[b], nrmv[b]

            def edge5(e5, _):
                for u in range(5):
                    e = e5 * 5 + u
                    nb = nb_ref[pl.ds(e * 16, 16)]
                    for jj in range(_D // 16):
                        t = tb[e, pl.ds(jj * 16, 16)]
                        tb[e, pl.ds(jj * 16, 16)] = t * nb
                return 0
            jax.lax.fori_loop(0, _CH // 5, edge5, 0)

        for d in idx3_descs(0, 0):
            d.start()
        for d in idx3_descs(1, 1):
            d.start()
        for d in idx3_descs(2, 2):
            d.start()
        dst_desc(0, 0).start()
        for d in idx3_descs(0, 0):
            d.wait()
        t_desc(0).start()
        t_desc(0).wait()
        radd_desc(0).start(add=True)
        for d in idx3_descs(1, 1):
            d.wait()
        t_desc(1).start()

        def outer(kb, _):
            for u in range(4):
                kk = kb * 4 + u
                b = u
                b1 = (u + 1) % 4
                b2s = (u + 2) % 4
                b3s = (u + 3) % 4
                radd_desc(b).wait()
                @pl.when(kk >= 2)
                def _():
                    scat_desc(b2s).wait()
                @pl.when(kk + 3 < _NCHUNK)
                def _():
                    for d in idx3_descs(kk + 3, b3s):
                        d.start()
                @pl.when(kk + 1 < _NCHUNK)
                def _():
                    dst_desc(kk + 1, b1).start()
                    t_desc(b1).wait()
                    radd_desc(b1).start(add=True)
                @pl.when(kk + 2 < _NCHUNK)
                def _():
                    for d in idx3_descs(kk + 2, b2s):
                        d.wait()
                    t_desc(b2s).start()
                compute(b)
                dst_desc(kk, b).wait()
                scat_desc(b).start(add=True)
            return 0
        jax.lax.fori_loop(0, _NCHUNK // 4, outer, 0)

        scat_desc((_NCHUNK - 2) % 4).wait()
        scat_desc((_NCHUNK - 1) % 4).wait()

        plsc.subcore_barrier()
        pltpu.make_async_copy(acc.at[pl.ds(s * _RPW, _RPW)],
                              out_hbm.at[c, pl.ds(s * _RPW, _RPW)],
                              ss0).start()
        pltpu.make_async_copy(acc.at[pl.ds(s * _RPW, _RPW)],
                              out_hbm.at[c, pl.ds(s * _RPW, _RPW)],
                              ss0).wait()

    return k(tcomb, rcomb, srcp, typep, dst, norm)


def kernel(x, rel_repr, edge_index, edge_type, edge_norm,
           in_w, out_w, loop_w, w_rel, loop_rel, bias, bn_gamma, bn_beta):
    half = _E // 2
    src = edge_index[0].astype(jnp.int32)
    dst = edge_index[1].astype(jnp.int32)
    shift = (jnp.arange(_E, dtype=jnp.int32) >= half).astype(jnp.int32)
    srcp = (src + shift * _N).reshape(_NW, _NCHUNK, _CH)
    typep = edge_type.astype(jnp.int32).reshape(_NW, _NCHUNK, _CH)
    dst3 = dst.reshape(_NW, _NCHUNK, _CH)
    norm16 = jnp.reshape(
        jnp.broadcast_to(edge_norm[:, None], (_E, 16)),
        (_NW, _NCHUNK, _CH * 16))

    tcomb = _node_tables(x, in_w, out_w)
    rcomb, rel_out = _rel_tables(rel_repr, in_w, out_w, w_rel)
    partials = _sc_edge_scatter(tcomb, rcomb, srcp, typep, dst3, norm16)
    out = _epilogue(partials, x, loop_w, loop_rel, bias, bn_gamma, bn_beta)
    return out, rel_out


# speedup vs baseline: 1.0122x; 1.0120x over previous
import functools

import jax
import jax.numpy as jnp
from jax.experimental import pallas as pl
from jax.experimental.pallas import tpu as pltpu
from jax.experimental.pallas import tpu_sc as plsc

_N = 10000
_E = 320000
_D = 128
_R = 200
_EPS = 1e-5

_NC = 2
_NS = 16
_NW = _NC * _NS
_EPT = _E // _NW
_CH = 50
_NCHUNK = _EPT // _CH
_NPAD = 10112
_RPW = _NPAD // _NS
_ZR = 8


def _tables_body(x_ref, w_ref, out_ref):
    out_ref[...] = jnp.dot(x_ref[...], w_ref[0],
                           preferred_element_type=jnp.float32,
                           precision=jax.lax.Precision.HIGHEST)


def _node_tables(x, in_w, out_w):
    w_stack = jnp.stack([in_w, out_w])
    nb = 10
    blk = _N // nb
    return pl.pallas_call(
        _tables_body,
        grid=(2, nb),
        in_specs=[
            pl.BlockSpec((blk, _D), lambda w, i: (i, 0)),
            pl.BlockSpec((1, _D, _D), lambda w, i: (w, 0, 0)),
        ],
        out_specs=pl.BlockSpec((blk, _D), lambda w, i: (w * nb + i, 0)),
        out_shape=jax.ShapeDtypeStruct((2 * _N, _D), jnp.float32),
    )(x, w_stack)


def _rel_body(rel_ref, inw_ref, outw_ref, wrel_ref, rcomb_ref, relout_ref):
    r = rel_ref[...]
    hi = jax.lax.Precision.HIGHEST
    rcomb_ref[pl.ds(0, _R), :] = -jnp.dot(r, inw_ref[...],
                                          preferred_element_type=jnp.float32,
                                          precision=hi)
    rcomb_ref[pl.ds(_R, _R), :] = -jnp.dot(r, outw_ref[...],
                                           preferred_element_type=jnp.float32,
                                           precision=hi)
    relout_ref[...] = jnp.dot(r, wrel_ref[...],
                              preferred_element_type=jnp.float32,
                              precision=hi)


def _rel_tables(rel, in_w, out_w, w_rel):
    return pl.pallas_call(
        _rel_body,
        out_shape=(
            jax.ShapeDtypeStruct((2 * _R, _D), jnp.float32),
            jax.ShapeDtypeStruct((_R, _D), jnp.float32),
        ),
    )(rel, in_w, out_w, w_rel)


def _epilogue_body(p_ref, x_ref, lw_ref, lrel_ref, bias_ref, g_ref, b_ref,
                   out_ref):
    agg = p_ref[0, pl.ds(0, _N)] + p_ref[1, pl.ds(0, _N)]
    loop_term = jnp.dot(x_ref[...] - lrel_ref[...], lw_ref[...],
                        preferred_element_type=jnp.float32,
                        precision=jax.lax.Precision.HIGHEST)
    h = (agg + loop_term) * (1.0 / 3.0) + bias_ref[...]
    mean = jnp.mean(h, axis=0, keepdims=True)
    var = jnp.mean((h - mean) ** 2, axis=0, keepdims=True)
    h = (h - mean) / jnp.sqrt(var + _EPS) * g_ref[...] + b_ref[...]
    out_ref[...] = jnp.maximum(h, 0.0)


def _epilogue(partials, x, loop_w, loop_rel, bias, bn_gamma, bn_beta):
    return pl.pallas_call(
        _epilogue_body,
        out_shape=jax.ShapeDtypeStruct((_N, _D), jnp.float32),
    )(partials, x, loop_w, loop_rel.reshape(1, _D), bias.reshape(1, _D),
      bn_gamma.reshape(1, _D), bn_beta.reshape(1, _D))


def _sc_edge_scatter(tcomb, rcomb, srcp, typep, dst, norm):
    mesh = plsc.VectorSubcoreMesh(core_axis_name="c", subcore_axis_name="s")

    @functools.partial(
        pl.kernel,
        out_type=jax.ShapeDtypeStruct((_NC, _NPAD, _D), jnp.float32),
        mesh=mesh,
        scratch_types=[
            pltpu.VMEM_SHARED((_NPAD, _D), jnp.float32),
            pltpu.VMEM_SHARED((_R, _D), jnp.float32),
            pltpu.VMEM((_CH,), jnp.int32),
            pltpu.VMEM((_CH,), jnp.int32),
            pltpu.VMEM((_CH,), jnp.int32),
            pltpu.VMEM((_CH,), jnp.int32),
            pltpu.VMEM((_CH,), jnp.int32),
            pltpu.VMEM((_CH,), jnp.int32),
            pltpu.VMEM((_CH,), jnp.int32),
            pltpu.VMEM((_CH,), jnp.int32),
            pltpu.VMEM((_CH,), jnp.int32),
            pltpu.VMEM((_CH,), jnp.int32),
            pltpu.VMEM((_CH,), jnp.int32),
            pltpu.VMEM((_CH,), jnp.int32),
            pltpu.VMEM((_CH * 16,), jnp.float32),
            pltpu.VMEM((_CH * 16,), jnp.float32),
            pltpu.VMEM((_CH * 16,), jnp.float32),
            pltpu.VMEM((_CH * 16,), jnp.float32),
            pltpu.VMEM((_CH, _D), jnp.float32),
            pltpu.VMEM((_CH, _D), jnp.float32),
            pltpu.VMEM((_CH, _D), jnp.float32),
            pltpu.VMEM((_CH, _D), jnp.float32),
            pltpu.SemaphoreType.DMA,
            pltpu.SemaphoreType.DMA,
            pltpu.SemaphoreType.DMA,
            pltpu.SemaphoreType.DMA,
            pltpu.SemaphoreType.DMA,
            pltpu.SemaphoreType.DMA,
            pltpu.SemaphoreType.DMA,
            pltpu.SemaphoreType.DMA,
            pltpu.SemaphoreType.DMA,
            pltpu.SemaphoreType.DMA,
            pltpu.SemaphoreType.DMA,
            pltpu.SemaphoreType.DMA,
            pltpu.SemaphoreType.DMA,
            pltpu.SemaphoreType.DMA,
            pltpu.SemaphoreType.DMA,
            pltpu.SemaphoreType.DMA,
        ],
    )
    def k(t_hbm, r_hbm, src_hbm, typ_hbm, dst_hbm, nrm_hbm, out_hbm,
          acc, rsp, sv0, sv1, sv2, sv3, tv0, tv1, tv2, tv3, dv0, dv1, dv2, dv3,
          nv0, nv1, nv2, nv3, t0, t1, t2, t3,
          si0, si1, si2, si3, st0, st1, st2, st3, sd0, sd1, sd2, sd3,
          ss0, ss1, ss2, ss3):
        c = jax.lax.axis_index("c")
        s = jax.lax.axis_index("s")
        wid = c * _NS + s
        srcv = (sv0, sv1, sv2, sv3)
        typv = (tv0, tv1, tv2, tv3)
        dstv = (dv0, dv1, dv2, dv3)
        nrmv = (nv0, nv1, nv2, nv3)
        trow = (t0, t1, t2, t3)
        semi = (si0, si1, si2, si3)
        semt = (st0, st1, st2, st3)
        semd = (sd0, sd1, sd2, sd3)
        sems = (ss0, ss1, ss2, ss3)

        zero16 = jnp.zeros((16,), jnp.float32)
        for i in range(8):
            for j in range(_D // 16):
                t0[i, pl.ds(j * 16, 16)] = zero16
        nz = _RPW // 8
        for i in range(nz):
            pltpu.make_async_copy(
                t0.at[pl.ds(0, 8)],
                acc.at[pl.ds(s * _RPW + i * 8, 8)], si0).start()
        for i in range(nz):
            pltpu.make_async_copy(
                t0.at[pl.ds(0, 8)],
                acc.at[pl.ds(s * _RPW + i * 8, 8)], si0).wait()
        @pl.when(s < _R // 40)
        def _():
            pltpu.sync_copy(r_hbm.at[pl.ds(c * _R + s * 40, 40)],
                            t0.at[pl.ds(0, 40)])
            pltpu.sync_copy(t0.at[pl.ds(0, 40)],
                            rsp.at[pl.ds(s * 40, 40)])
        plsc.subcore_barrier()

        def idx3_descs(j, b):
            return (
                pltpu.make_async_copy(src_hbm.at[wid, j], srcv[b], semi[b]),
                pltpu.make_async_copy(typ_hbm.at[wid, j], typv[b], semi[b]),
                pltpu.make_async_copy(nrm_hbm.at[wid, j], nrmv[b], semi[b]),
            )

        def dst_desc(j, b):
            return pltpu.make_async_copy(dst_hbm.at[wid, j], dstv[b], semd[b])

        def t_desc(b):
            return pltpu.make_async_copy(t_hbm.at[srcv[b]], trow[b], semt[b])

        def radd_desc(b):
            return pltpu.make_async_copy(rsp.at[typv[b]], trow[b], semt[b])

        def scat_desc(b):
            return pltpu.make_async_copy(trow[b], acc.at[dstv[b]], sems[b])

        def compute(b):
            tb, nb_ref = trow[b], nrmv[b]

            def edge5(e5, _):
                for u in range(5):
                    e = e5 * 5 + u
                    nb = nb_ref[pl.ds(e * 16, 16)]
                    for jj in range(_D // 16):
                        t = tb[e, pl.ds(jj * 16, 16)]
                        tb[e, pl.ds(jj * 16, 16)] = t * nb
                return 0
            jax.lax.fori_loop(0, _CH // 5, edge5, 0)

        for d in idx3_descs(0, 0):
            d.start()
        for d in idx3_descs(1, 1):
            d.start()
        for d in idx3_descs(2, 2):
            d.start()
        dst_desc(0, 0).start()
        for d in idx3_descs(0, 0):
            d.wait()
        t_desc(0).start()
        t_desc(0).wait()
        radd_desc(0).start(add=True)
        for d in idx3_descs(1, 1):
            d.wait()
        t_desc(1).start()

        def outer(kb, _):
            for u in range(4):
                kk = kb * 4 + u
                b = u
                b1 = (u + 1) % 4
                b2s = (u + 2) % 4
                b3s = (u + 3) % 4
                radd_desc(b).wait()
                @pl.when(kk >= 2)
                def _():
                    scat_desc(b2s).wait()
                @pl.when(kk + 3 < _NCHUNK)
                def _():
                    for d in idx3_descs(kk + 3, b3s):
                        d.start()
                @pl.when(kk + 1 < _NCHUNK)
                def _():
                    dst_desc(kk + 1, b1).start()
                    t_desc(b1).wait()
                    radd_desc(b1).start(add=True)
                @pl.when(kk + 2 < _NCHUNK)
                def _():
                    for d in idx3_descs(kk + 2, b2s):
                        d.wait()
                    t_desc(b2s).start()
                compute(b)
                dst_desc(kk, b).wait()
                scat_desc(b).start(add=True)
            return 0
        jax.lax.fori_loop(0, _NCHUNK // 4, outer, 0)

        scat_desc((_NCHUNK - 2) % 4).wait()
        scat_desc((_NCHUNK - 1) % 4).wait()

        plsc.subcore_barrier()
        pltpu.make_async_copy(acc.at[pl.ds(s * _RPW, _RPW)],
                              out_hbm.at[c, pl.ds(s * _RPW, _RPW)],
                              ss0).start()
        pltpu.make_async_copy(acc.at[pl.ds(s * _RPW, _RPW)],
                              out_hbm.at[c, pl.ds(s * _RPW, _RPW)],
                              ss0).wait()

    return k(tcomb, rcomb, srcp, typep, dst, norm)


def kernel(x, rel_repr, edge_index, edge_type, edge_norm,
           in_w, out_w, loop_w, w_rel, loop_rel, bias, bn_gamma, bn_beta):
    half = _E // 2
    src = edge_index[0].astype(jnp.int32)
    dst = edge_index[1].astype(jnp.int32)
    shift = (jnp.arange(_E, dtype=jnp.int32) >= half).astype(jnp.int32)
    srcp = (src + shift * _N).reshape(_NW, _NCHUNK, _CH)
    typep = edge_type.astype(jnp.int32).reshape(_NW, _NCHUNK, _CH)
    dst3 = dst.reshape(_NW, _NCHUNK, _CH)
    norm16 = jnp.reshape(
        jnp.broadcast_to(edge_norm[:, None], (_E, 16)),
        (_NW, _NCHUNK, _CH * 16))

    tcomb = _node_tables(x, in_w, out_w)
    rcomb, rel_out = _rel_tables(rel_repr, in_w, out_w, w_rel)
    partials = _sc_edge_scatter(tcomb, rcomb, srcp, typep, dst3, norm16)
    out = _epilogue(partials, x, loop_w, loop_rel, bias, bn_gamma, bn_beta)
    return out, rel_out
